# baseline (speedup 1.0000x reference)
import numpy as np
import jax
import jax.numpy as jnp
from jax import lax
from jax.experimental import pallas as pl
from jax.experimental.pallas import tpu as pltpu
from jax.experimental.pallas import tpu_sc as plsc

_N_LEVELS = 16
_FPL = 2
_PER_LEVEL_SCALE = 1.38
_TABLE_SIZE = 1 << 19
_BASE_RES = 16
_N_POINTS = 262144
_MASK = _TABLE_SIZE - 1
_P1 = int(np.int32(np.uint32(2654435761)))
_P2 = int(np.int32(np.uint32(805459861)))

_L = 16
_NW = 32
_C = 512
_OW = _N_LEVELS * _FPL


def _body(xyzt_hbm, table_hbm, out_hbm,
          x_v, ia_a, ib_a, ia_b, ib_b, wc_a, wc_b,
          f0_a, f1_a, f0_b, f1_b, acc_v,
          s0_a, s1_a, s0_b, s1_b):
    nc = 2
    wid = lax.axis_index("s") * nc + lax.axis_index("c")
    ppw = _N_POINTS // _NW
    nchunks = ppw // _C
    lanes = lax.iota(jnp.int32, _L)
    ngroups = _C // _L
    ia_bufs = (ia_a, ia_b)
    ib_bufs = (ib_a, ib_b)
    wc_bufs = (wc_a, wc_b)
    f0_bufs = (f0_a, f0_b)
    f1_bufs = (f1_a, f1_b)
    s0 = (s0_a, s0_b)
    s1 = (s1_a, s1_b)

    def start_gather(buf):
        pltpu.async_copy(table_hbm.at[ia_bufs[buf]], f0_bufs[buf], s0[buf])
        pltpu.async_copy(table_hbm.at[ib_bufs[buf]], f1_bufs[buf], s1[buf])

    def wait_gather(buf):
        pltpu.make_async_copy(table_hbm.at[ia_bufs[buf]], f0_bufs[buf],
                              s0[buf]).wait()
        pltpu.make_async_copy(table_hbm.at[ib_bufs[buf]], f1_bufs[buf],
                              s1[buf]).wait()

    def phase1(l, buf):
        scale = float(_BASE_RES) * (_PER_LEVEL_SCALE ** l) - 1.0
        a = jnp.float32(scale * 0.5)
        b = jnp.float32(scale * 0.5 + 0.5)
        lbase2 = l * _TABLE_SIZE * 2
        ia_v = ia_bufs[buf]
        ib_v = ib_bufs[buf]
        wc_v = wc_bufs[buf]

        def p1(g, carry):
            off = g * _L
            xs = x_v[pl.ds(off, _L)]
            ys = x_v[pl.ds(_C + off, _L)]
            zs = x_v[pl.ds(2 * _C + off, _L)]
            px = xs * a + b
            py = ys * a + b
            pz = zs * a + b
            ix = px.astype(jnp.int32)
            iy = py.astype(jnp.int32)
            iz = pz.astype(jnp.int32)
            wx = px - ix.astype(jnp.float32)
            wy = py - iy.astype(jnp.float32)
            wz = pz - iz.astype(jnp.float32)
            hxs = (ix, ix + 1)
            hy0 = iy * _P1
            hys = (hy0, hy0 + _P1)
            hz0 = iz * _P2
            hzs = (hz0, hz0 + _P2)
            wxs = (1.0 - wx, wx)
            wy1 = wy
            wy0 = 1.0 - wy
            wz1 = wz
            wz0 = 1.0 - wz
            tyz = (wy0 * wz0, wy1 * wz0, wy0 * wz1, wy1 * wz1)
            for c in range(8):
                bx = c & 1
                h = hxs[bx] ^ hys[(c >> 1) & 1] ^ hzs[(c >> 2) & 1]
                e0 = ((h & _MASK) << 1) + lbase2
                ia_v[pl.ds(c * _C + off, _L)] = e0
                ib_v[pl.ds(c * _C + off, _L)] = e0 + 1
                wc_v[pl.ds(c * _C + off, _L)] = wxs[bx] * tyz[c >> 1]
            return carry

        lax.fori_loop(0, ngroups, p1, 0, unroll=False)

    def phase2(l, buf):
        f0_v = f0_bufs[buf]
        f1_v = f1_bufs[buf]
        wc_v = wc_bufs[buf]

        def p2(g, carry):
            off = g * _L
            pt = off + lanes
            acc0 = jnp.zeros((_L,), jnp.float32)
            acc1 = jnp.zeros((_L,), jnp.float32)
            for c in range(8):
                f0 = f0_v[pl.ds(c * _C + off, _L)]
                f1 = f1_v[pl.ds(c * _C + off, _L)]
                w = wc_v[pl.ds(c * _C + off, _L)]
                acc0 = acc0 + w * f0
                acc1 = acc1 + w * f1
            o = pt * _OW + (2 * l)
            plsc.store_scatter(acc_v, [o], acc0)
            plsc.store_scatter(acc_v, [o + 1], acc1)
            return carry

        lax.fori_loop(0, ngroups, p2, 0, unroll=False)

    def chunk_body(k, carry):
        base = wid * ppw + k * _C
        for d in range(3):
            pltpu.sync_copy(xyzt_hbm.at[pl.ds(d * _N_POINTS + base, _C)],
                            x_v.at[pl.ds(d * _C, _C)])
        phase1(0, 0)
        start_gather(0)
        for l in range(1, _N_LEVELS):
            buf = l % 2
            pbuf = 1 - buf
            phase1(l, buf)
            start_gather(buf)
            wait_gather(pbuf)
            phase2(l - 1, pbuf)
        lbuf = (_N_LEVELS - 1) % 2
        wait_gather(lbuf)
        phase2(_N_LEVELS - 1, lbuf)
        pltpu.sync_copy(acc_v, out_hbm.at[pl.ds(base * _OW, _C * _OW)])
        return carry

    lax.fori_loop(0, nchunks, chunk_body, 0, unroll=False)


@jax.jit
def _encode(xyzt, table_flat):
    mesh = plsc.VectorSubcoreMesh(core_axis_name="c", subcore_axis_name="s")
    f = pl.kernel(
        _body,
        out_type=jax.ShapeDtypeStruct((_N_POINTS * _OW,), jnp.float32),
        mesh=mesh,
        scratch_types=[
            pltpu.VMEM((3 * _C,), jnp.float32),
            pltpu.VMEM((8 * _C,), jnp.int32),
            pltpu.VMEM((8 * _C,), jnp.int32),
            pltpu.VMEM((8 * _C,), jnp.int32),
            pltpu.VMEM((8 * _C,), jnp.int32),
            pltpu.VMEM((8 * _C,), jnp.float32),
            pltpu.VMEM((8 * _C,), jnp.float32),
            pltpu.VMEM((8 * _C,), jnp.float32),
            pltpu.VMEM((8 * _C,), jnp.float32),
            pltpu.VMEM((8 * _C,), jnp.float32),
            pltpu.VMEM((8 * _C,), jnp.float32),
            pltpu.VMEM((_C * _OW,), jnp.float32),
            pltpu.SemaphoreType.DMA,
            pltpu.SemaphoreType.DMA,
            pltpu.SemaphoreType.DMA,
            pltpu.SemaphoreType.DMA,
        ],
        compiler_params=pltpu.CompilerParams(needs_layout_passes=False),
    )
    return f(xyzt, table_flat)


def kernel(xyz, table):
    xyzt = xyz.T.reshape(3 * _N_POINTS)
    table_flat = table.reshape(_N_LEVELS * _TABLE_SIZE * _FPL)
    out = _encode(xyzt, table_flat)
    return out.reshape(_N_POINTS, _OW)


# speedup vs baseline: 4.9202x; 4.9202x over previous
import numpy as np
import jax
import jax.numpy as jnp
from jax import lax
from jax.experimental import pallas as pl
from jax.experimental.pallas import tpu as pltpu
from jax.experimental.pallas import tpu_sc as plsc

_N_LEVELS = 16
_FPL = 2
_PER_LEVEL_SCALE = 1.38
_TABLE_SIZE = 1 << 19
_BASE_RES = 16
_N_POINTS = 262144
_MASK = _TABLE_SIZE - 1
_P1 = int(np.int32(np.uint32(2654435761)))
_P2 = int(np.int32(np.uint32(805459861)))

_L = 16
_NW = 32
_C = 512
_OW = _N_LEVELS * _FPL


def _body(xyzt_hbm, t0_hbm, t1_hbm, out_hbm,
          x_v, ia_a, ia_b, wc_a, wc_b,
          f0_a, f1_a, f0_b, f1_b, acc_v,
          s0_a, s1_a, s0_b, s1_b):
    nc = 2
    wid = lax.axis_index("s") * nc + lax.axis_index("c")
    ppw = _N_POINTS // _NW
    nchunks = ppw // _C
    lanes = lax.iota(jnp.int32, _L)
    ngroups = _C // _L
    ia_bufs = (ia_a, ia_b)
    wc_bufs = (wc_a, wc_b)
    f0_bufs = (f0_a, f0_b)
    f1_bufs = (f1_a, f1_b)
    s0 = (s0_a, s0_b)
    s1 = (s1_a, s1_b)

    def start_gather(buf):
        pltpu.async_copy(t0_hbm.at[ia_bufs[buf]], f0_bufs[buf], s0[buf])
        pltpu.async_copy(t1_hbm.at[ia_bufs[buf]], f1_bufs[buf], s1[buf])

    def wait_gather(buf):
        pltpu.make_async_copy(t0_hbm.at[ia_bufs[buf]], f0_bufs[buf],
                              s0[buf]).wait()
        pltpu.make_async_copy(t1_hbm.at[ia_bufs[buf]], f1_bufs[buf],
                              s1[buf]).wait()

    def phase1(l, buf):
        scale = float(_BASE_RES) * (_PER_LEVEL_SCALE ** l) - 1.0
        a = jnp.float32(scale * 0.5)
        b = jnp.float32(scale * 0.5 + 0.5)
        lbase = l * _TABLE_SIZE
        ia_v = ia_bufs[buf]
        wc_v = wc_bufs[buf]

        def p1(g, carry):
            off = g * _L
            xs = x_v[pl.ds(off, _L)]
            ys = x_v[pl.ds(_C + off, _L)]
            zs = x_v[pl.ds(2 * _C + off, _L)]
            px = xs * a + b
            py = ys * a + b
            pz = zs * a + b
            ix = px.astype(jnp.int32)
            iy = py.astype(jnp.int32)
            iz = pz.astype(jnp.int32)
            wx = px - ix.astype(jnp.float32)
            wy = py - iy.astype(jnp.float32)
            wz = pz - iz.astype(jnp.float32)
            hxs = (ix, ix + 1)
            hy0 = iy * _P1
            hys = (hy0, hy0 + _P1)
            hz0 = iz * _P2
            hzs = (hz0, hz0 + _P2)
            wxs = (1.0 - wx, wx)
            wy1 = wy
            wy0 = 1.0 - wy
            wz1 = wz
            wz0 = 1.0 - wz
            tyz = (wy0 * wz0, wy1 * wz0, wy0 * wz1, wy1 * wz1)
            for c in range(8):
                bx = c & 1
                h = hxs[bx] ^ hys[(c >> 1) & 1] ^ hzs[(c >> 2) & 1]
                ia_v[pl.ds(c * _C + off, _L)] = (h & _MASK) + lbase
                wc_v[pl.ds(c * _C + off, _L)] = wxs[bx] * tyz[c >> 1]
            return carry

        lax.fori_loop(0, ngroups, p1, 0, unroll=False)

    def phase2(l, buf):
        f0_v = f0_bufs[buf]
        f1_v = f1_bufs[buf]
        wc_v = wc_bufs[buf]

        def p2(g, carry):
            off = g * _L
            pt = off + lanes
            acc0 = jnp.zeros((_L,), jnp.float32)
            acc1 = jnp.zeros((_L,), jnp.float32)
            for c in range(8):
                f0 = f0_v[pl.ds(c * _C + off, _L)]
                f1 = f1_v[pl.ds(c * _C + off, _L)]
                w = wc_v[pl.ds(c * _C + off, _L)]
                acc0 = acc0 + w * f0
                acc1 = acc1 + w * f1
            o = pt * _OW + (2 * l)
            plsc.store_scatter(acc_v, [o], acc0)
            plsc.store_scatter(acc_v, [o + 1], acc1)
            return carry

        lax.fori_loop(0, ngroups, p2, 0, unroll=False)

    def chunk_body(k, carry):
        base = wid * ppw + k * _C
        for d in range(3):
            pltpu.sync_copy(xyzt_hbm.at[pl.ds(d * _N_POINTS + base, _C)],
                            x_v.at[pl.ds(d * _C, _C)])
        phase1(0, 0)
        start_gather(0)
        for l in range(1, _N_LEVELS):
            buf = l % 2
            pbuf = 1 - buf
            phase1(l, buf)
            start_gather(buf)
            wait_gather(pbuf)
            phase2(l - 1, pbuf)
        lbuf = (_N_LEVELS - 1) % 2
        wait_gather(lbuf)
        phase2(_N_LEVELS - 1, lbuf)
        pltpu.sync_copy(acc_v, out_hbm.at[pl.ds(base * _OW, _C * _OW)])
        return carry

    lax.fori_loop(0, nchunks, chunk_body, 0, unroll=False)


@jax.jit
def _encode(xyzt, t0, t1):
    mesh = plsc.VectorSubcoreMesh(core_axis_name="c", subcore_axis_name="s")
    f = pl.kernel(
        _body,
        out_type=jax.ShapeDtypeStruct((_N_POINTS * _OW,), jnp.float32),
        mesh=mesh,
        scratch_types=[
            pltpu.VMEM((3 * _C,), jnp.float32),
            pltpu.VMEM((8 * _C,), jnp.int32),
            pltpu.VMEM((8 * _C,), jnp.int32),
            pltpu.VMEM((8 * _C,), jnp.float32),
            pltpu.VMEM((8 * _C,), jnp.float32),
            pltpu.VMEM((8 * _C,), jnp.float32),
            pltpu.VMEM((8 * _C,), jnp.float32),
            pltpu.VMEM((8 * _C,), jnp.float32),
            pltpu.VMEM((8 * _C,), jnp.float32),
            pltpu.VMEM((_C * _OW,), jnp.float32),
            pltpu.SemaphoreType.DMA,
            pltpu.SemaphoreType.DMA,
            pltpu.SemaphoreType.DMA,
            pltpu.SemaphoreType.DMA,
        ],
        compiler_params=pltpu.CompilerParams(needs_layout_passes=False),
    )
    return f(xyzt, t0, t1)


def kernel(xyz, table):
    xyzt = xyz.T.reshape(3 * _N_POINTS)
    t0 = table[:, :, 0].reshape(_N_LEVELS * _TABLE_SIZE)
    t1 = table[:, :, 1].reshape(_N_LEVELS * _TABLE_SIZE)
    out = _encode(xyzt, t0, t1)
    return out.reshape(_N_POINTS, _OW)


# speedup vs baseline: 5.7564x; 1.1699x over previous
import numpy as np
import jax
import jax.numpy as jnp
from jax import lax
from jax.experimental import pallas as pl
from jax.experimental.pallas import tpu as pltpu
from jax.experimental.pallas import tpu_sc as plsc

_N_LEVELS = 16
_FPL = 2
_PER_LEVEL_SCALE = 1.38
_TABLE_SIZE = 1 << 19
_BASE_RES = 16
_N_POINTS = 262144
_MASK = _TABLE_SIZE - 1
_P1 = int(np.int32(np.uint32(2654435761)))
_P2 = int(np.int32(np.uint32(805459861)))

_L = 16
_NW = 32
_C = 256
_OW = _N_LEVELS * _FPL


def _scale(l):
    return float(_BASE_RES) * (_PER_LEVEL_SCALE ** l) - 1.0


_CACHED = 3
_G = [int(np.floor(_scale(l) + 0.5)) + 2 for l in range(_CACHED)]
_SEC = 1024
_GPAD = [(g * g * g + _SEC - 1) // _SEC * _SEC for g in _G]


def _body(xyzt_hbm, t0_hbm, t1_hbm, out_hbm,
          x_v, ia_a, ia_b, wc_a, wc_b,
          f0_a, f1_a, f0_b, f1_b, acc_v,
          g0_0, g1_0, g0_1, g1_1, g0_2, g1_2,
          s0_a, s1_a, s0_b, s1_b):
    nc = 2
    wid = lax.axis_index("s") * nc + lax.axis_index("c")
    ppw = _N_POINTS // _NW
    nchunks = ppw // _C
    lanes = lax.iota(jnp.int32, _L)
    ngroups = _C // _L
    ia_bufs = (ia_a, ia_b)
    wc_bufs = (wc_a, wc_b)
    f0_bufs = (f0_a, f0_b)
    f1_bufs = (f1_a, f1_b)
    s0 = (s0_a, s0_b)
    s1 = (s1_a, s1_b)
    grids = ((g0_0, g1_0), (g0_1, g1_1), (g0_2, g1_2))

    def build_grid(l, g0_v, g1_v):
        G = _G[l]
        G2 = G * G
        nsec = _GPAD[l] // _SEC
        lbase = l * _TABLE_SIZE
        inv2 = jnp.float32(1.0 / G2)
        inv1 = jnp.float32(1.0 / G)

        def sec_body(s, carry):
            se = lax.rem(s + wid, nsec)
            sbase = se * _SEC

            def gen(i, carry2):
                lin = sbase + i * _L + lanes
                linf = lin.astype(jnp.float32) + 0.5
                cz = (linf * inv2).astype(jnp.int32)
                rem = lin - cz * G2
                remf = rem.astype(jnp.float32) + 0.5
                cy = (remf * inv1).astype(jnp.int32)
                cx = rem - cy * G
                h = cx ^ (cy * _P1) ^ (cz * _P2)
                ia_a[pl.ds(i * _L, _L)] = (h & _MASK) + lbase
                return carry2

            lax.fori_loop(0, _SEC // _L, gen, 0, unroll=False)
            iref = ia_a.at[pl.ds(0, _SEC)]
            c0 = pltpu.async_copy(t0_hbm.at[iref],
                                  g0_v.at[pl.ds(sbase, _SEC)], s0_a)
            c1 = pltpu.async_copy(t1_hbm.at[iref],
                                  g1_v.at[pl.ds(sbase, _SEC)], s1_a)
            c0.wait()
            c1.wait()
            return carry

        lax.fori_loop(0, nsec, sec_body, 0, unroll=False)

    def cached_level(l):
        G = _G[l]
        G2 = G * G
        g0_v, g1_v = grids[l]
        scale = _scale(l)
        a = jnp.float32(scale * 0.5)
        b = jnp.float32(scale * 0.5 + 0.5)
        coffs = (0, 1, G, G + 1, G2, G2 + 1, G2 + G, G2 + G + 1)

        def pc(g, carry):
            off = g * _L
            xs = x_v[pl.ds(off, _L)]
            ys = x_v[pl.ds(_C + off, _L)]
            zs = x_v[pl.ds(2 * _C + off, _L)]
            px = xs * a + b
            py = ys * a + b
            pz = zs * a + b
            ix = px.astype(jnp.int32)
            iy = py.astype(jnp.int32)
            iz = pz.astype(jnp.int32)
            wx = px - ix.astype(jnp.float32)
            wy = py - iy.astype(jnp.float32)
            wz = pz - iz.astype(jnp.float32)
            b0 = ix + iy * G + iz * G2
            wxs = (1.0 - wx, wx)
            wy1 = wy
            wy0 = 1.0 - wy
            wz1 = wz
            wz0 = 1.0 - wz
            tyz = (wy0 * wz0, wy1 * wz0, wy0 * wz1, wy1 * wz1)
            acc0 = jnp.zeros((_L,), jnp.float32)
            acc1 = jnp.zeros((_L,), jnp.float32)
            for c in range(8):
                gi = b0 + coffs[c] if c else b0
                f0 = plsc.load_gather(g0_v, [gi])
                f1 = plsc.load_gather(g1_v, [gi])
                w = wxs[c & 1] * tyz[c >> 1]
                acc0 = acc0 + w * f0
                acc1 = acc1 + w * f1
            pt = off + lanes
            o = pt * _OW + (2 * l)
            plsc.store_scatter(acc_v, [o], acc0)
            plsc.store_scatter(acc_v, [o + 1], acc1)
            return carry

        lax.fori_loop(0, ngroups, pc, 0, unroll=False)

    def start_gather(buf):
        pltpu.async_copy(t0_hbm.at[ia_bufs[buf]], f0_bufs[buf], s0[buf])
        pltpu.async_copy(t1_hbm.at[ia_bufs[buf]], f1_bufs[buf], s1[buf])

    def wait_gather(buf):
        pltpu.make_async_copy(t0_hbm.at[ia_bufs[buf]], f0_bufs[buf],
                              s0[buf]).wait()
        pltpu.make_async_copy(t1_hbm.at[ia_bufs[buf]], f1_bufs[buf],
                              s1[buf]).wait()

    def phase1(l, buf):
        scale = _scale(l)
        a = jnp.float32(scale * 0.5)
        b = jnp.float32(scale * 0.5 + 0.5)
        lbase = l * _TABLE_SIZE
        ia_v = ia_bufs[buf]
        wc_v = wc_bufs[buf]

        def p1(g, carry):
            off = g * _L
            xs = x_v[pl.ds(off, _L)]
            ys = x_v[pl.ds(_C + off, _L)]
            zs = x_v[pl.ds(2 * _C + off, _L)]
            px = xs * a + b
            py = ys * a + b
            pz = zs * a + b
            ix = px.astype(jnp.int32)
            iy = py.astype(jnp.int32)
            iz = pz.astype(jnp.int32)
            wx = px - ix.astype(jnp.float32)
            wy = py - iy.astype(jnp.float32)
            wz = pz - iz.astype(jnp.float32)
            hxs = (ix, ix + 1)
            hy0 = iy * _P1
            hys = (hy0, hy0 + _P1)
            hz0 = iz * _P2
            hzs = (hz0, hz0 + _P2)
            wxs = (1.0 - wx, wx)
            wy1 = wy
            wy0 = 1.0 - wy
            wz1 = wz
            wz0 = 1.0 - wz
            tyz = (wy0 * wz0, wy1 * wz0, wy0 * wz1, wy1 * wz1)
            for c in range(8):
                bx = c & 1
                h = hxs[bx] ^ hys[(c >> 1) & 1] ^ hzs[(c >> 2) & 1]
                ia_v[pl.ds(c * _C + off, _L)] = (h & _MASK) + lbase
                wc_v[pl.ds(c * _C + off, _L)] = wxs[bx] * tyz[c >> 1]
            return carry

        lax.fori_loop(0, ngroups, p1, 0, unroll=False)

    def phase2(l, buf):
        f0_v = f0_bufs[buf]
        f1_v = f1_bufs[buf]
        wc_v = wc_bufs[buf]

        def p2(g, carry):
            off = g * _L
            pt = off + lanes
            acc0 = jnp.zeros((_L,), jnp.float32)
            acc1 = jnp.zeros((_L,), jnp.float32)
            for c in range(8):
                f0 = f0_v[pl.ds(c * _C + off, _L)]
                f1 = f1_v[pl.ds(c * _C + off, _L)]
                w = wc_v[pl.ds(c * _C + off, _L)]
                acc0 = acc0 + w * f0
                acc1 = acc1 + w * f1
            o = pt * _OW + (2 * l)
            plsc.store_scatter(acc_v, [o], acc0)
            plsc.store_scatter(acc_v, [o + 1], acc1)
            return carry

        lax.fori_loop(0, ngroups, p2, 0, unroll=False)

    for l in range(_CACHED):
        build_grid(l, *grids[l])

    def chunk_body(k, carry):
        base = wid * ppw + k * _C
        for d in range(3):
            pltpu.sync_copy(xyzt_hbm.at[pl.ds(d * _N_POINTS + base, _C)],
                            x_v.at[pl.ds(d * _C, _C)])
        phase1(_CACHED, 0)
        start_gather(0)
        for l in range(_CACHED):
            cached_level(l)
        for l in range(_CACHED + 1, _N_LEVELS):
            buf = (l - _CACHED) % 2
            pbuf = 1 - buf
            phase1(l, buf)
            start_gather(buf)
            wait_gather(pbuf)
            phase2(l - 1, pbuf)
        lbuf = (_N_LEVELS - 1 - _CACHED) % 2
        wait_gather(lbuf)
        phase2(_N_LEVELS - 1, lbuf)
        pltpu.sync_copy(acc_v, out_hbm.at[pl.ds(base * _OW, _C * _OW)])
        return carry

    lax.fori_loop(0, nchunks, chunk_body, 0, unroll=False)


@jax.jit
def _encode(xyzt, t0, t1):
    mesh = plsc.VectorSubcoreMesh(core_axis_name="c", subcore_axis_name="s")
    f = pl.kernel(
        _body,
        out_type=jax.ShapeDtypeStruct((_N_POINTS * _OW,), jnp.float32),
        mesh=mesh,
        scratch_types=[
            pltpu.VMEM((3 * _C,), jnp.float32),
            pltpu.VMEM((8 * _C,), jnp.int32),
            pltpu.VMEM((8 * _C,), jnp.int32),
            pltpu.VMEM((8 * _C,), jnp.float32),
            pltpu.VMEM((8 * _C,), jnp.float32),
            pltpu.VMEM((8 * _C,), jnp.float32),
            pltpu.VMEM((8 * _C,), jnp.float32),
            pltpu.VMEM((8 * _C,), jnp.float32),
            pltpu.VMEM((8 * _C,), jnp.float32),
            pltpu.VMEM((_C * _OW,), jnp.float32),
            pltpu.VMEM((_GPAD[0],), jnp.float32),
            pltpu.VMEM((_GPAD[0],), jnp.float32),
            pltpu.VMEM((_GPAD[1],), jnp.float32),
            pltpu.VMEM((_GPAD[1],), jnp.float32),
            pltpu.VMEM((_GPAD[2],), jnp.float32),
            pltpu.VMEM((_GPAD[2],), jnp.float32),
            pltpu.SemaphoreType.DMA,
            pltpu.SemaphoreType.DMA,
            pltpu.SemaphoreType.DMA,
            pltpu.SemaphoreType.DMA,
        ],
        compiler_params=pltpu.CompilerParams(needs_layout_passes=False),
    )
    return f(xyzt, t0, t1)


def kernel(xyz, table):
    xyzt = xyz.T.reshape(3 * _N_POINTS)
    t0 = table[:, :, 0].reshape(_N_LEVELS * _TABLE_SIZE)
    t1 = table[:, :, 1].reshape(_N_LEVELS * _TABLE_SIZE)
    out = _encode(xyzt, t0, t1)
    return out.reshape(_N_POINTS, _OW)


# speedup vs baseline: 9.7325x; 1.6907x over previous
import numpy as np
import jax
import jax.numpy as jnp
from jax import lax
from jax.experimental import pallas as pl
from jax.experimental.pallas import tpu as pltpu
from jax.experimental.pallas import tpu_sc as plsc

_N_LEVELS = 16
_FPL = 2
_PER_LEVEL_SCALE = 1.38
_TABLE_SIZE = 1 << 19
_BASE_RES = 16
_N_POINTS = 262144
_MASK = _TABLE_SIZE - 1
_P1 = int(np.int32(np.uint32(2654435761)))
_P2 = int(np.int32(np.uint32(805459861)))

_L = 16
_NW = 32
_C = 256
_OW = _N_LEVELS * _FPL


def _scale(l):
    return float(_BASE_RES) * (_PER_LEVEL_SCALE ** l) - 1.0


_CACHED = 3
_G = [int(np.floor(_scale(l) + 0.5)) + 2 for l in range(_CACHED)]
_SEC = 1024
_GPAD = [(g * g * g + _SEC - 1) // _SEC * _SEC for g in _G]


def _unpack2(v):
    bf = plsc.bitcast(v, jnp.bfloat16)
    return plsc.unpack(bf, format=plsc.PackFormat.INTERLEAVED)


def _body(xyzt_hbm, tw_hbm, out_hbm,
          x_v, ia_a, ia_b, wc_a, wc_b,
          fp_a, fp_b, acc_v,
          g_0, g_1, g_2,
          s_a, s_b):
    nc = 2
    wid = lax.axis_index("s") * nc + lax.axis_index("c")
    ppw = _N_POINTS // _NW
    nchunks = ppw // _C
    lanes = lax.iota(jnp.int32, _L)
    ngroups = _C // _L
    ia_bufs = (ia_a, ia_b)
    wc_bufs = (wc_a, wc_b)
    fp_bufs = (fp_a, fp_b)
    sems = (s_a, s_b)
    grids = (g_0, g_1, g_2)

    def build_grid(l, g_v):
        G = _G[l]
        G2 = G * G
        nsec = _GPAD[l] // _SEC
        lbase = l * _TABLE_SIZE
        inv2 = jnp.float32(1.0 / G2)
        inv1 = jnp.float32(1.0 / G)

        def sec_body(s, carry):
            se = lax.rem(s + wid, nsec)
            sbase = se * _SEC

            def gen(i, carry2):
                lin = sbase + i * _L + lanes
                linf = lin.astype(jnp.float32) + 0.5
                cz = (linf * inv2).astype(jnp.int32)
                rem = lin - cz * G2
                remf = rem.astype(jnp.float32) + 0.5
                cy = (remf * inv1).astype(jnp.int32)
                cx = rem - cy * G
                h = cx ^ (cy * _P1) ^ (cz * _P2)
                ia_a[pl.ds(i * _L, _L)] = (h & _MASK) + lbase
                return carry2

            lax.fori_loop(0, _SEC // _L, gen, 0, unroll=False)
            iref = ia_a.at[pl.ds(0, _SEC)]
            pltpu.async_copy(tw_hbm.at[iref],
                             g_v.at[pl.ds(sbase, _SEC)], s_a).wait()
            return carry

        lax.fori_loop(0, nsec, sec_body, 0, unroll=False)

    def cached_level(l):
        G = _G[l]
        G2 = G * G
        g_v = grids[l]
        scale = _scale(l)
        a = jnp.float32(scale * 0.5)
        b = jnp.float32(scale * 0.5 + 0.5)
        coffs = (0, 1, G, G + 1, G2, G2 + 1, G2 + G, G2 + G + 1)

        def pc(g, carry):
            off = g * _L
            xs = x_v[pl.ds(off, _L)]
            ys = x_v[pl.ds(_C + off, _L)]
            zs = x_v[pl.ds(2 * _C + off, _L)]
            px = xs * a + b
            py = ys * a + b
            pz = zs * a + b
            ix = px.astype(jnp.int32)
            iy = py.astype(jnp.int32)
            iz = pz.astype(jnp.int32)
            wx = px - ix.astype(jnp.float32)
            wy = py - iy.astype(jnp.float32)
            wz = pz - iz.astype(jnp.float32)
            b0 = ix + iy * G + iz * G2
            wxs = (1.0 - wx, wx)
            wy1 = wy
            wy0 = 1.0 - wy
            wz1 = wz
            wz0 = 1.0 - wz
            tyz = (wy0 * wz0, wy1 * wz0, wy0 * wz1, wy1 * wz1)
            acc0 = jnp.zeros((_L,), jnp.float32)
            acc1 = jnp.zeros((_L,), jnp.float32)
            for c in range(8):
                gi = b0 + coffs[c] if c else b0
                v = plsc.load_gather(g_v, [gi])
                f0, f1 = _unpack2(v)
                w = wxs[c & 1] * tyz[c >> 1]
                acc0 = acc0 + w * f0
                acc1 = acc1 + w * f1
            pt = off + lanes
            o = pt * _OW + (2 * l)
            plsc.store_scatter(acc_v, [o], acc0)
            plsc.store_scatter(acc_v, [o + 1], acc1)
            return carry

        lax.fori_loop(0, ngroups, pc, 0, unroll=False)

    def start_gather(buf):
        pltpu.async_copy(tw_hbm.at[ia_bufs[buf]], fp_bufs[buf], sems[buf])

    def wait_gather(buf):
        pltpu.make_async_copy(tw_hbm.at[ia_bufs[buf]], fp_bufs[buf],
                              sems[buf]).wait()

    def phase1(l, buf):
        scale = _scale(l)
        a = jnp.float32(scale * 0.5)
        b = jnp.float32(scale * 0.5 + 0.5)
        lbase = l * _TABLE_SIZE
        ia_v = ia_bufs[buf]
        wc_v = wc_bufs[buf]

        def p1(g, carry):
            off = g * _L
            xs = x_v[pl.ds(off, _L)]
            ys = x_v[pl.ds(_C + off, _L)]
            zs = x_v[pl.ds(2 * _C + off, _L)]
            px = xs * a + b
            py = ys * a + b
            pz = zs * a + b
            ix = px.astype(jnp.int32)
            iy = py.astype(jnp.int32)
            iz = pz.astype(jnp.int32)
            wx = px - ix.astype(jnp.float32)
            wy = py - iy.astype(jnp.float32)
            wz = pz - iz.astype(jnp.float32)
            hxs = (ix, ix + 1)
            hy0 = iy * _P1
            hys = (hy0, hy0 + _P1)
            hz0 = iz * _P2
            hzs = (hz0, hz0 + _P2)
            wxs = (1.0 - wx, wx)
            wy1 = wy
            wy0 = 1.0 - wy
            wz1 = wz
            wz0 = 1.0 - wz
            tyz = (wy0 * wz0, wy1 * wz0, wy0 * wz1, wy1 * wz1)
            for c in range(8):
                bx = c & 1
                h = hxs[bx] ^ hys[(c >> 1) & 1] ^ hzs[(c >> 2) & 1]
                ia_v[pl.ds(c * _C + off, _L)] = (h & _MASK) + lbase
                wc_v[pl.ds(c * _C + off, _L)] = wxs[bx] * tyz[c >> 1]
            return carry

        lax.fori_loop(0, ngroups, p1, 0, unroll=False)

    def phase2(l, buf):
        fp_v = fp_bufs[buf]
        wc_v = wc_bufs[buf]

        def p2(g, carry):
            off = g * _L
            pt = off + lanes
            acc0 = jnp.zeros((_L,), jnp.float32)
            acc1 = jnp.zeros((_L,), jnp.float32)
            for c in range(8):
                v = fp_v[pl.ds(c * _C + off, _L)]
                f0, f1 = _unpack2(v)
                w = wc_v[pl.ds(c * _C + off, _L)]
                acc0 = acc0 + w * f0
                acc1 = acc1 + w * f1
            o = pt * _OW + (2 * l)
            plsc.store_scatter(acc_v, [o], acc0)
            plsc.store_scatter(acc_v, [o + 1], acc1)
            return carry

        lax.fori_loop(0, ngroups, p2, 0, unroll=False)

    for l in range(_CACHED):
        build_grid(l, grids[l])

    def chunk_body(k, carry):
        base = wid * ppw + k * _C
        for d in range(3):
            pltpu.sync_copy(xyzt_hbm.at[pl.ds(d * _N_POINTS + base, _C)],
                            x_v.at[pl.ds(d * _C, _C)])
        phase1(_CACHED, 0)
        start_gather(0)
        for l in range(_CACHED):
            cached_level(l)
        for l in range(_CACHED + 1, _N_LEVELS):
            buf = (l - _CACHED) % 2
            pbuf = 1 - buf
            phase1(l, buf)
            start_gather(buf)
            wait_gather(pbuf)
            phase2(l - 1, pbuf)
        lbuf = (_N_LEVELS - 1 - _CACHED) % 2
        wait_gather(lbuf)
        phase2(_N_LEVELS - 1, lbuf)
        pltpu.sync_copy(acc_v, out_hbm.at[pl.ds(base * _OW, _C * _OW)])
        return carry

    lax.fori_loop(0, nchunks, chunk_body, 0, unroll=False)


@jax.jit
def _encode(xyzt, tw):
    mesh = plsc.VectorSubcoreMesh(core_axis_name="c", subcore_axis_name="s")
    f = pl.kernel(
        _body,
        out_type=jax.ShapeDtypeStruct((_N_POINTS * _OW,), jnp.float32),
        mesh=mesh,
        scratch_types=[
            pltpu.VMEM((3 * _C,), jnp.float32),
            pltpu.VMEM((8 * _C,), jnp.int32),
            pltpu.VMEM((8 * _C,), jnp.int32),
            pltpu.VMEM((8 * _C,), jnp.float32),
            pltpu.VMEM((8 * _C,), jnp.float32),
            pltpu.VMEM((8 * _C,), jnp.int32),
            pltpu.VMEM((8 * _C,), jnp.int32),
            pltpu.VMEM((_C * _OW,), jnp.float32),
            pltpu.VMEM((_GPAD[0],), jnp.int32),
            pltpu.VMEM((_GPAD[1],), jnp.int32),
            pltpu.VMEM((_GPAD[2],), jnp.int32),
            pltpu.SemaphoreType.DMA,
            pltpu.SemaphoreType.DMA,
        ],
        compiler_params=pltpu.CompilerParams(needs_layout_passes=False),
    )
    return f(xyzt, tw)


def kernel(xyz, table):
    xyzt = xyz.T.reshape(3 * _N_POINTS)
    t0 = table[:, :, 0].reshape(_N_LEVELS * _TABLE_SIZE)
    t1 = table[:, :, 1].reshape(_N_LEVELS * _TABLE_SIZE)
    b0 = lax.bitcast_convert_type(t0.astype(jnp.bfloat16), jnp.uint16)
    b1 = lax.bitcast_convert_type(t1.astype(jnp.bfloat16), jnp.uint16)
    tw = (b0.astype(jnp.uint32) | (b1.astype(jnp.uint32) << 16))
    out = _encode(xyzt, lax.bitcast_convert_type(tw, jnp.int32))
    return out.reshape(_N_POINTS, _OW)
